# Initial kernel scaffold; baseline (speedup 1.0000x reference)
import jax
import jax.numpy as jnp
from jax.experimental import pallas as pl


def kernel(z, codebooks):
    raise NotImplementedError("write your pallas kernel here")


# speedup vs baseline: 1.3667x; 1.3667x over previous
import jax
import jax.numpy as jnp
from jax.experimental import pallas as pl
from jax.experimental.pallas import tpu as pltpu

_N_EMBED = 1024
_EMBED_DIM = 256
_NUM_Q = 4
_BLK = 1152


def _core(r, insq, cbsq, cbt_ref, w3_ref):
    r16 = r.astype(jnp.bfloat16)
    mm = jnp.dot(r16, cbt_ref[...], preferred_element_type=jnp.float32)
    dist = insq + cbsq - 2.0 * mm
    m = jnp.min(dist, axis=1, keepdims=True)
    iota = jax.lax.broadcasted_iota(jnp.int32, (_BLK, _N_EMBED), 1)
    idx = jnp.min(jnp.where(dist == m, iota, _N_EMBED),
                  axis=1, keepdims=True)
    oh = (iota == idx).astype(jnp.bfloat16)
    oh3 = jnp.concatenate([oh, oh, oh], axis=1)
    q = jnp.dot(oh3, w3_ref[...], preferred_element_type=jnp.float32)
    return idx, q


def _first_body(r_ref, insq_ref, cbsq_ref, cbt_ref, w3_ref,
                rout_ref, qt_ref, idx_ref):
    r = r_ref[...]
    idx, q = _core(r, insq_ref[...], cbsq_ref[...], cbt_ref, w3_ref)
    rout_ref[...] = r - q
    qt_ref[...] = q
    idx_ref[...] = idx


def _mid_body(r_ref, insq_ref, cbsq_ref, cbt_ref, w3_ref,
              qt_in_ref, rout_ref, qt_ref, idx_ref):
    r = r_ref[...]
    idx, q = _core(r, insq_ref[...], cbsq_ref[...], cbt_ref, w3_ref)
    rout_ref[...] = r - q
    qt_ref[...] = qt_in_ref[...] + q
    idx_ref[...] = idx


def _last_body(r_ref, insq_ref, cbsq_ref, cbt_ref, w3_ref,
               qt_in_ref, z_ref, zq_ref, idx_ref, loss_ref):
    r = r_ref[...]
    idx, q = _core(r, insq_ref[...], cbsq_ref[...], cbt_ref, w3_ref)
    quant = qt_in_ref[...] + q
    zb = z_ref[...]
    zq_ref[...] = zb + (quant - zb)
    idx_ref[...] = idx
    loss_ref[...] = jnp.broadcast_to(jnp.sum((zb - quant) ** 2), (1, 8, 128))


def _tok_spec():
    return pl.BlockSpec((_BLK, _EMBED_DIM), lambda i: (i, 0))


def _col_spec():
    return pl.BlockSpec((_BLK, 1), lambda i: (i, 0))


_CONST_SPECS = [
    pl.BlockSpec((1, _N_EMBED), lambda i: (0, 0)),
    pl.BlockSpec((_EMBED_DIM, _N_EMBED), lambda i: (0, 0)),
    pl.BlockSpec((3 * _N_EMBED, _EMBED_DIM), lambda i: (0, 0)),
]
_PARAMS = pltpu.CompilerParams(dimension_semantics=("parallel",))


def kernel(z, codebooks):
    n_tok = z.shape[0] * z.shape[1]
    grid = (n_tok // _BLK,)
    z_flat = z.reshape(n_tok, _EMBED_DIM)
    f32 = jnp.float32

    cbt16 = codebooks.transpose(0, 2, 1).astype(jnp.bfloat16)
    mask = jnp.uint32(0xFFFF0000)
    u = jax.lax.bitcast_convert_type(codebooks, jnp.uint32)
    hi32 = jax.lax.bitcast_convert_type(u & mask, f32)
    r1 = codebooks - hi32
    u1 = jax.lax.bitcast_convert_type(r1, jnp.uint32)
    mid32 = jax.lax.bitcast_convert_type(u1 & mask, f32)
    lo32 = r1 - mid32
    hi = hi32.astype(jnp.bfloat16)
    mid = mid32.astype(jnp.bfloat16)
    lo = lo32.astype(jnp.bfloat16)
    w3 = jnp.concatenate([hi, mid, lo], axis=1)

    tok_f32 = jax.ShapeDtypeStruct((n_tok, _EMBED_DIM), f32)
    idx_shape = jax.ShapeDtypeStruct((n_tok, 1), jnp.int32)

    r = z_flat
    quant = None
    codes_cols = []
    for l in range(_NUM_Q):
        cb_l = codebooks[l]
        cb_sq = jnp.sum(cb_l * cb_l, axis=1)[None, :]
        in_sq = jnp.sum(r * r, axis=1, keepdims=True)
        consts = (cb_sq, cbt16[l], w3[l])
        if l == 0:
            r, quant, idx = pl.pallas_call(
                _first_body, grid=grid,
                in_specs=[_tok_spec(), _col_spec(), *_CONST_SPECS],
                out_specs=[_tok_spec(), _tok_spec(), _col_spec()],
                out_shape=[tok_f32, tok_f32, idx_shape],
                compiler_params=_PARAMS,
            )(r, in_sq, *consts)
        elif l < _NUM_Q - 1:
            r, quant, idx = pl.pallas_call(
                _mid_body, grid=grid,
                in_specs=[_tok_spec(), _col_spec(), *_CONST_SPECS,
                          _tok_spec()],
                out_specs=[_tok_spec(), _tok_spec(), _col_spec()],
                out_shape=[tok_f32, tok_f32, idx_shape],
                compiler_params=_PARAMS,
            )(r, in_sq, *consts, quant)
        else:
            z_q, idx, loss_parts = pl.pallas_call(
                _last_body, grid=grid,
                in_specs=[_tok_spec(), _col_spec(), *_CONST_SPECS,
                          _tok_spec(), _tok_spec()],
                out_specs=[_tok_spec(), _col_spec(),
                           pl.BlockSpec((1, 8, 128), lambda i: (i, 0, 0))],
                out_shape=[tok_f32, idx_shape,
                           jax.ShapeDtypeStruct((grid[0], 8, 128), f32)],
                compiler_params=_PARAMS,
            )(r, in_sq, *consts, quant, z_flat)
        codes_cols.append(idx)

    codes = jnp.concatenate(codes_cols, axis=1).reshape(
        z.shape[0], z.shape[1], _NUM_Q)
    z_q = z_q.reshape(z.shape)
    commit_loss = jnp.sum(loss_parts[:, 0, 0]) / (n_tok * _EMBED_DIM)
    return z_q, codes, commit_loss


# speedup vs baseline: 1.3855x; 1.0138x over previous
import jax
import jax.numpy as jnp
from jax.experimental import pallas as pl
from jax.experimental.pallas import tpu as pltpu

_N_EMBED = 1024
_EMBED_DIM = 256
_NUM_Q = 4
_BLK = 1152


def _core(r, insq, cbsq, cbt_ref, w3_ref):
    r16 = r.astype(jnp.bfloat16)
    mm = jnp.dot(r16, cbt_ref[...], preferred_element_type=jnp.float32)
    dist = insq + cbsq - 2.0 * mm
    m = jnp.min(dist, axis=1, keepdims=True)
    iota = jax.lax.broadcasted_iota(jnp.int32, (_BLK, _N_EMBED), 1)
    idx = jnp.min(jnp.where(dist == m, iota, _N_EMBED),
                  axis=1, keepdims=True)
    oh = (iota == idx).astype(jnp.bfloat16)
    oh3 = jnp.concatenate([oh, oh, oh], axis=1)
    q = jnp.dot(oh3, w3_ref[...], preferred_element_type=jnp.float32)
    return idx, q


def _level_body(r_ref, insq_ref, cbsq_ref, cbt_ref, w3_ref,
                rout_ref, idx_ref):
    r = r_ref[...]
    idx, q = _core(r, insq_ref[...], cbsq_ref[...], cbt_ref, w3_ref)
    rout_ref[...] = r - q
    idx_ref[...] = idx


def _last_body(r_ref, insq_ref, cbsq_ref, cbt_ref, w3_ref,
               z_ref, zq_ref, idx_ref, loss_ref):
    r = r_ref[...]
    idx, q = _core(r, insq_ref[...], cbsq_ref[...], cbt_ref, w3_ref)
    r4 = r - q
    zb = z_ref[...]
    quant = zb - r4
    zq_ref[...] = zb + (quant - zb)
    idx_ref[...] = idx
    loss_ref[...] = jnp.broadcast_to(jnp.sum((zb - quant) ** 2), (1, 8, 128))


def _tok_spec():
    return pl.BlockSpec((_BLK, _EMBED_DIM), lambda i: (i, 0))


def _col_spec():
    return pl.BlockSpec((_BLK, 1), lambda i: (i, 0))


_CONST_SPECS = [
    pl.BlockSpec((1, _N_EMBED), lambda i: (0, 0)),
    pl.BlockSpec((_EMBED_DIM, _N_EMBED), lambda i: (0, 0)),
    pl.BlockSpec((3 * _N_EMBED, _EMBED_DIM), lambda i: (0, 0)),
]
_PARAMS = pltpu.CompilerParams(dimension_semantics=("parallel",))


def kernel(z, codebooks):
    n_tok = z.shape[0] * z.shape[1]
    grid = (n_tok // _BLK,)
    z_flat = z.reshape(n_tok, _EMBED_DIM)
    f32 = jnp.float32

    cbt16 = codebooks.transpose(0, 2, 1).astype(jnp.bfloat16)
    mask = jnp.uint32(0xFFFF0000)
    u = jax.lax.bitcast_convert_type(codebooks, jnp.uint32)
    hi32 = jax.lax.bitcast_convert_type(u & mask, f32)
    r1 = codebooks - hi32
    u1 = jax.lax.bitcast_convert_type(r1, jnp.uint32)
    mid32 = jax.lax.bitcast_convert_type(u1 & mask, f32)
    lo32 = r1 - mid32
    hi = hi32.astype(jnp.bfloat16)
    mid = mid32.astype(jnp.bfloat16)
    lo = lo32.astype(jnp.bfloat16)
    w3 = jnp.concatenate([hi, mid, lo], axis=1)

    tok_f32 = jax.ShapeDtypeStruct((n_tok, _EMBED_DIM), f32)
    idx_shape = jax.ShapeDtypeStruct((n_tok, 1), jnp.int32)

    r = z_flat
    codes_cols = []
    for l in range(_NUM_Q):
        cb_l = codebooks[l]
        cb_sq = jnp.sum(cb_l * cb_l, axis=1)[None, :]
        in_sq = jnp.sum(r * r, axis=1, keepdims=True)
        consts = (cb_sq, cbt16[l], w3[l])
        if l < _NUM_Q - 1:
            r, idx = pl.pallas_call(
                _level_body, grid=grid,
                in_specs=[_tok_spec(), _col_spec(), *_CONST_SPECS],
                out_specs=[_tok_spec(), _col_spec()],
                out_shape=[tok_f32, idx_shape],
                compiler_params=_PARAMS,
            )(r, in_sq, *consts)
        else:
            z_q, idx, loss_parts = pl.pallas_call(
                _last_body, grid=grid,
                in_specs=[_tok_spec(), _col_spec(), *_CONST_SPECS,
                          _tok_spec()],
                out_specs=[_tok_spec(), _col_spec(),
                           pl.BlockSpec((1, 8, 128), lambda i: (i, 0, 0))],
                out_shape=[tok_f32, idx_shape,
                           jax.ShapeDtypeStruct((grid[0], 8, 128), f32)],
                compiler_params=_PARAMS,
            )(r, in_sq, *consts, z_flat)
        codes_cols.append(idx)

    codes = jnp.concatenate(codes_cols, axis=1).reshape(
        z.shape[0], z.shape[1], _NUM_Q)
    z_q = z_q.reshape(z.shape)
    commit_loss = jnp.sum(loss_parts[:, 0, 0]) / (n_tok * _EMBED_DIM)
    return z_q, codes, commit_loss


# speedup vs baseline: 1.4274x; 1.0302x over previous
import functools

import jax
import jax.numpy as jnp
from jax import lax
from jax.experimental import pallas as pl
from jax.experimental.pallas import tpu as pltpu
from jax.experimental.pallas import tpu_sc as plsc

_N_EMBED = 1024
_EMBED_DIM = 256
_NUM_Q = 4
_BLK = 1152


def _argmin_body(r_ref, insq_ref, cbsq_ref, cbt_ref, idx_ref):
    r = r_ref[...]
    r16 = r.astype(jnp.bfloat16)
    mm = jnp.dot(r16, cbt_ref[...], preferred_element_type=jnp.float32)
    dist = insq_ref[...] + cbsq_ref[...] - 2.0 * mm
    m = jnp.min(dist, axis=1, keepdims=True)
    iota = jax.lax.broadcasted_iota(jnp.int32, (_BLK, _N_EMBED), 1)
    idx_ref[...] = jnp.min(jnp.where(dist == m, iota, _N_EMBED),
                           axis=1, keepdims=True)


def _final_body(z_ref, r3_ref, q3_ref, zq_ref, loss_ref):
    zb = z_ref[...]
    r4 = r3_ref[...] - q3_ref[...]
    quant = zb - r4
    zq_ref[...] = zb + (quant - zb)
    loss_ref[...] = jnp.broadcast_to(jnp.sum((zb - quant) ** 2), (1, 8, 128))


def _sc_gather(table, idx, n_tok):
    info = plsc.get_sparse_core_info()
    nw = info.num_cores * info.num_subcores
    b_per_w = n_tok // nw
    mesh = plsc.VectorSubcoreMesh(core_axis_name="c", subcore_axis_name="s")

    @functools.partial(
        pl.kernel, mesh=mesh,
        out_type=jax.ShapeDtypeStruct((n_tok, _EMBED_DIM), jnp.float32),
        scratch_types=[
            pltpu.VMEM((b_per_w,), jnp.int32),
            pltpu.VMEM((b_per_w, _EMBED_DIM), jnp.float32),
            pltpu.SemaphoreType.DMA,
        ],
    )
    def gk(table_hbm, idx_hbm, out_hbm, idx_v, rows_v, sem):
        wid = lax.axis_index("s") * info.num_cores + lax.axis_index("c")
        base = wid * b_per_w
        pltpu.sync_copy(idx_hbm.at[pl.ds(base, b_per_w)], idx_v)
        pltpu.async_copy(table_hbm.at[idx_v], rows_v, sem).wait()
        pltpu.sync_copy(rows_v, out_hbm.at[pl.ds(base, b_per_w)])

    return gk(table, idx)


def _tok_spec():
    return pl.BlockSpec((_BLK, _EMBED_DIM), lambda i: (i, 0))


def _col_spec():
    return pl.BlockSpec((_BLK, 1), lambda i: (i, 0))


_PARAMS = pltpu.CompilerParams(dimension_semantics=("parallel",))


def kernel(z, codebooks):
    n_tok = z.shape[0] * z.shape[1]
    grid = (n_tok // _BLK,)
    z_flat = z.reshape(n_tok, _EMBED_DIM)
    f32 = jnp.float32

    cbt16 = codebooks.transpose(0, 2, 1).astype(jnp.bfloat16)
    idx_shape = jax.ShapeDtypeStruct((n_tok, 1), jnp.int32)

    r = z_flat
    codes_cols = []
    for l in range(_NUM_Q):
        cb_l = codebooks[l]
        cb_sq = jnp.sum(cb_l * cb_l, axis=1)[None, :]
        in_sq = jnp.sum(r * r, axis=1, keepdims=True)
        idx = pl.pallas_call(
            _argmin_body, grid=grid,
            in_specs=[_tok_spec(), _col_spec(),
                      pl.BlockSpec((1, _N_EMBED), lambda i: (0, 0)),
                      pl.BlockSpec((_EMBED_DIM, _N_EMBED), lambda i: (0, 0))],
            out_specs=_col_spec(),
            out_shape=idx_shape,
            compiler_params=_PARAMS,
        )(r, in_sq, cb_sq, cbt16[l])
        codes_cols.append(idx)
        q = _sc_gather(cb_l, idx.reshape(n_tok), n_tok)
        if l < _NUM_Q - 1:
            r = r - q
        else:
            z_q, loss_parts = pl.pallas_call(
                _final_body, grid=grid,
                in_specs=[_tok_spec(), _tok_spec(), _tok_spec()],
                out_specs=[_tok_spec(),
                           pl.BlockSpec((1, 8, 128), lambda i: (i, 0, 0))],
                out_shape=[jax.ShapeDtypeStruct((n_tok, _EMBED_DIM), f32),
                           jax.ShapeDtypeStruct((grid[0], 8, 128), f32)],
                compiler_params=_PARAMS,
            )(z_flat, r, q)

    codes = jnp.concatenate(codes_cols, axis=1).reshape(
        z.shape[0], z.shape[1], _NUM_Q)
    z_q = z_q.reshape(z.shape)
    commit_loss = jnp.sum(loss_parts[:, 0, 0]) / (n_tok * _EMBED_DIM)
    return z_q, codes, commit_loss
